# initial kernel scaffold (unmeasured)
import jax
import jax.numpy as jnp
from jax import lax
from jax.experimental import pallas as pl
from jax.experimental.pallas import tpu as pltpu

N_DEV = 16
M_PER = 256


def kernel(x, w_mat):
    m_tot, k_per = x.shape
    _, n = w_mat.shape

    def body(x_ref, w_ref, out_ref, send_buf, comm_buf,
             send_sems, recv_sems, credit_sem):
        my = lax.axis_index("i")
        left = lax.rem(my + N_DEV - 1, N_DEV)
        right = lax.rem(my + 1, N_DEV)

        barrier_sem = pltpu.get_barrier_semaphore()
        for nbr in (left, right):
            pl.semaphore_signal(
                barrier_sem, inc=1,
                device_id=(nbr,), device_id_type=pl.DeviceIdType.MESH,
            )
        pl.semaphore_wait(barrier_sem, 2)

        def partial_chunk(c):
            rows = x_ref[pl.ds(c * M_PER, M_PER), :]
            return jnp.dot(rows, w_ref[:, :],
                           preferred_element_type=jnp.float32)

        c0 = lax.rem(my + N_DEV - 1, N_DEV)
        send_buf[0, :, :] = partial_chunk(c0)

        rdmas = []
        for s in range(N_DEV - 1):
            slot = s % 2
            if s >= 2:
                pl.semaphore_wait(credit_sem, 1)
            rdma = pltpu.make_async_remote_copy(
                src_ref=send_buf.at[slot],
                dst_ref=comm_buf.at[slot],
                send_sem=send_sems.at[slot],
                recv_sem=recv_sems.at[slot],
                device_id=(right,),
                device_id_type=pl.DeviceIdType.MESH,
            )
            rdma.start()
            rdmas.append(rdma)

            c = lax.rem(my + (N_DEV - 2 - s), N_DEV)
            nxt = partial_chunk(c)

            rdma.wait_recv()
            if s < N_DEV - 2:
                rdmas[s - 1].wait_send() if s >= 1 else None
                send_buf[(s + 1) % 2, :, :] = comm_buf[slot, :, :] + nxt
            else:
                out_ref[:, :] = comm_buf[slot, :, :] + nxt
            if s <= N_DEV - 4:
                pl.semaphore_signal(
                    credit_sem, inc=1,
                    device_id=(left,), device_id_type=pl.DeviceIdType.MESH,
                )

        rdmas[N_DEV - 3].wait_send()
        rdmas[N_DEV - 2].wait_send()

    return pl.pallas_call(
        body,
        out_shape=jax.ShapeDtypeStruct((M_PER, n), jnp.float32),
        in_specs=[
            pl.BlockSpec(memory_space=pltpu.VMEM),
            pl.BlockSpec(memory_space=pltpu.VMEM),
        ],
        out_specs=pl.BlockSpec(memory_space=pltpu.VMEM),
        scratch_shapes=[
            pltpu.VMEM((2, M_PER, n), jnp.float32),
            pltpu.VMEM((2, M_PER, n), jnp.float32),
            pltpu.SemaphoreType.DMA((2,)),
            pltpu.SemaphoreType.DMA((2,)),
            pltpu.SemaphoreType.REGULAR,
        ],
        compiler_params=pltpu.CompilerParams(collective_id=0),
    )(x, w_mat)


# baseline (device time: 376825 ns/iter reference)
import jax
import jax.numpy as jnp
from jax import lax
from jax.experimental import pallas as pl
from jax.experimental.pallas import tpu as pltpu

N_DEV = 16
M_PER = 256


def kernel(x, w_mat):
    m_tot, k_per = x.shape
    _, n = w_mat.shape

    def body(x_ref, w_ref, out_ref, send_buf, comm_buf,
             send_sems, recv_sems, credit_sem):
        my = lax.axis_index("i")
        left = lax.rem(my + N_DEV - 1, N_DEV)
        right = lax.rem(my + 1, N_DEV)

        barrier_sem = pltpu.get_barrier_semaphore()
        for nbr in (left, right):
            pl.semaphore_signal(
                barrier_sem, inc=1,
                device_id=(nbr,), device_id_type=pl.DeviceIdType.MESH,
            )
        pl.semaphore_wait(barrier_sem, 2)

        def partial_chunk(c):
            rows = x_ref[pl.ds(c * M_PER, M_PER), :]
            return jnp.dot(rows, w_ref[:, :],
                           preferred_element_type=jnp.float32)

        c0 = lax.rem(my + N_DEV - 1, N_DEV)
        send_buf[0, :, :] = partial_chunk(c0)

        rdmas = []
        for s in range(N_DEV - 1):
            slot = s % 2
            if s >= 2:
                pl.semaphore_wait(credit_sem, 1)
            rdma = pltpu.make_async_remote_copy(
                src_ref=send_buf.at[slot],
                dst_ref=comm_buf.at[slot],
                send_sem=send_sems.at[slot],
                recv_sem=recv_sems.at[slot],
                device_id=(right,),
                device_id_type=pl.DeviceIdType.MESH,
            )
            rdma.start()
            rdmas.append(rdma)

            c = lax.rem(my + (N_DEV - 2 - s), N_DEV)
            nxt = partial_chunk(c)

            rdma.wait_recv()
            if s < N_DEV - 2:
                if s >= 1:
                    rdmas[s - 1].wait_send()
                send_buf[(s + 1) % 2, :, :] = comm_buf[slot, :, :] + nxt
            else:
                out_ref[:, :] = comm_buf[slot, :, :] + nxt
            if s <= N_DEV - 4:
                pl.semaphore_signal(
                    credit_sem, inc=1,
                    device_id=(left,), device_id_type=pl.DeviceIdType.MESH,
                )

        rdmas[N_DEV - 3].wait_send()
        rdmas[N_DEV - 2].wait_send()

    return pl.pallas_call(
        body,
        out_shape=jax.ShapeDtypeStruct((M_PER, n), jnp.float32),
        in_specs=[
            pl.BlockSpec(memory_space=pltpu.VMEM),
            pl.BlockSpec(memory_space=pltpu.VMEM),
        ],
        out_specs=pl.BlockSpec(memory_space=pltpu.VMEM),
        scratch_shapes=[
            pltpu.VMEM((2, M_PER, n), jnp.float32),
            pltpu.VMEM((2, M_PER, n), jnp.float32),
            pltpu.SemaphoreType.DMA((2,)),
            pltpu.SemaphoreType.DMA((2,)),
            pltpu.SemaphoreType.REGULAR,
        ],
        compiler_params=pltpu.CompilerParams(collective_id=0),
    )(x, w_mat)


# device time: 228420 ns/iter; 1.6497x vs baseline; 1.6497x over previous
import jax
import jax.numpy as jnp
from jax import lax
from jax.experimental import pallas as pl
from jax.experimental.pallas import tpu as pltpu

N_DEV = 16
M_PER = 256


def kernel(x, w_mat):
    m_tot, k_per = x.shape
    _, n = w_mat.shape
    nh = n // 2

    def body(x_ref, w_ref, out_ref,
             send_r, send_l, comm_r, comm_l,
             send_sems_r, recv_sems_r, send_sems_l, recv_sems_l,
             credit_r, credit_l):
        my = lax.axis_index("i")
        left = lax.rem(my + N_DEV - 1, N_DEV)
        right = lax.rem(my + 1, N_DEV)

        barrier_sem = pltpu.get_barrier_semaphore()
        for nbr in (left, right):
            pl.semaphore_signal(
                barrier_sem, inc=1,
                device_id=(nbr,), device_id_type=pl.DeviceIdType.MESH,
            )
        pl.semaphore_wait(barrier_sem, 2)

        def partial_r(c):
            rows = x_ref[pl.ds(c * M_PER, M_PER), :]
            return jnp.dot(rows, w_ref[:, :nh],
                           preferred_element_type=jnp.float32)

        def partial_l(c):
            rows = x_ref[pl.ds(c * M_PER, M_PER), :]
            return jnp.dot(rows, w_ref[:, nh:],
                           preferred_element_type=jnp.float32)

        send_r[0, :, :] = partial_r(lax.rem(my + N_DEV - 1, N_DEV))
        send_l[0, :, :] = partial_l(lax.rem(my + 1, N_DEV))

        rdmas_r = []
        rdmas_l = []
        for s in range(N_DEV - 1):
            slot = s % 2
            if s >= 2:
                pl.semaphore_wait(credit_r, 1)
                pl.semaphore_wait(credit_l, 1)
            rdma_r = pltpu.make_async_remote_copy(
                src_ref=send_r.at[slot],
                dst_ref=comm_r.at[slot],
                send_sem=send_sems_r.at[slot],
                recv_sem=recv_sems_r.at[slot],
                device_id=(right,),
                device_id_type=pl.DeviceIdType.MESH,
            )
            rdma_l = pltpu.make_async_remote_copy(
                src_ref=send_l.at[slot],
                dst_ref=comm_l.at[slot],
                send_sem=send_sems_l.at[slot],
                recv_sem=recv_sems_l.at[slot],
                device_id=(left,),
                device_id_type=pl.DeviceIdType.MESH,
            )
            rdma_r.start()
            rdma_l.start()
            rdmas_r.append(rdma_r)
            rdmas_l.append(rdma_l)

            c_r = lax.rem(my + (N_DEV - 2 - s), N_DEV)
            c_l = lax.rem(my + 2 + s, N_DEV)
            nxt_r = partial_r(c_r)
            nxt_l = partial_l(c_l)

            rdma_r.wait_recv()
            rdma_l.wait_recv()
            if s < N_DEV - 2:
                if s >= 1:
                    rdmas_r[s - 1].wait_send()
                    rdmas_l[s - 1].wait_send()
                send_r[(s + 1) % 2, :, :] = comm_r[slot, :, :] + nxt_r
                send_l[(s + 1) % 2, :, :] = comm_l[slot, :, :] + nxt_l
            else:
                out_ref[:, :nh] = comm_r[slot, :, :] + nxt_r
                out_ref[:, nh:] = comm_l[slot, :, :] + nxt_l
            if s <= N_DEV - 4:
                pl.semaphore_signal(
                    credit_r, inc=1,
                    device_id=(left,), device_id_type=pl.DeviceIdType.MESH,
                )
                pl.semaphore_signal(
                    credit_l, inc=1,
                    device_id=(right,), device_id_type=pl.DeviceIdType.MESH,
                )

        for r in (rdmas_r, rdmas_l):
            r[N_DEV - 3].wait_send()
            r[N_DEV - 2].wait_send()

    return pl.pallas_call(
        body,
        out_shape=jax.ShapeDtypeStruct((M_PER, n), jnp.float32),
        in_specs=[
            pl.BlockSpec(memory_space=pltpu.VMEM),
            pl.BlockSpec(memory_space=pltpu.VMEM),
        ],
        out_specs=pl.BlockSpec(memory_space=pltpu.VMEM),
        scratch_shapes=[
            pltpu.VMEM((2, M_PER, nh), jnp.float32),
            pltpu.VMEM((2, M_PER, nh), jnp.float32),
            pltpu.VMEM((2, M_PER, nh), jnp.float32),
            pltpu.VMEM((2, M_PER, nh), jnp.float32),
            pltpu.SemaphoreType.DMA((2,)),
            pltpu.SemaphoreType.DMA((2,)),
            pltpu.SemaphoreType.DMA((2,)),
            pltpu.SemaphoreType.DMA((2,)),
            pltpu.SemaphoreType.REGULAR,
            pltpu.SemaphoreType.REGULAR,
        ],
        compiler_params=pltpu.CompilerParams(collective_id=0),
    )(x, w_mat)


# device time: 183747 ns/iter; 2.0508x vs baseline; 1.2431x over previous
import jax
import jax.numpy as jnp
from jax import lax
from jax.experimental import pallas as pl
from jax.experimental.pallas import tpu as pltpu

N_DEV = 16
M_PER = 256
N_SUB = 512


def kernel(x, w_mat):
    m_tot, k_per = x.shape
    _, n = w_mat.shape
    nh = n // 2

    def body(x_ref, w_ref, out_ref, *scratch):
        my = lax.axis_index("i")
        left = lax.rem(my + N_DEV - 1, N_DEV)
        right = lax.rem(my + 1, N_DEV)

        rings = []
        for q in range(4):
            is_right = q < 2
            rings.append(dict(
                send_buf=scratch[q],
                comm_buf=scratch[4 + q],
                send_sems=scratch[8 + q],
                recv_sems=scratch[12 + q],
                credit=scratch[16 + q],
                dst=right if is_right else left,
                upstream=left if is_right else right,
                col0=q * N_SUB,
                rdmas=[],
            ))

        barrier_sem = pltpu.get_barrier_semaphore()
        for nbr in (left, right):
            pl.semaphore_signal(
                barrier_sem, inc=1,
                device_id=(nbr,), device_id_type=pl.DeviceIdType.MESH,
            )
        pl.semaphore_wait(barrier_sem, 2)

        def partial(c, lo, hi):
            rows = x_ref[pl.ds(c * M_PER, M_PER), :]
            return jnp.dot(rows, w_ref[:, lo:hi],
                           preferred_element_type=jnp.float32)

        def start_hop(ring, s):
            slot = s % 2
            if s >= 2:
                pl.semaphore_wait(ring["credit"], 1)
            rdma = pltpu.make_async_remote_copy(
                src_ref=ring["send_buf"].at[slot],
                dst_ref=ring["comm_buf"].at[slot],
                send_sem=ring["send_sems"].at[slot],
                recv_sem=ring["recv_sems"].at[slot],
                device_id=(ring["dst"],),
                device_id_type=pl.DeviceIdType.MESH,
            )
            rdma.start()
            ring["rdmas"].append(rdma)

        seed_r = partial(lax.rem(my + N_DEV - 1, N_DEV), 0, nh)
        seed_l = partial(lax.rem(my + 1, N_DEV), nh, n)
        for q, ring in enumerate(rings):
            seed = seed_r if q < 2 else seed_l
            lo = ring["col0"] - (0 if q < 2 else nh)
            ring["send_buf"][0, :, :] = seed[:, lo:lo + N_SUB]
        for ring in rings:
            start_hop(ring, 0)

        for s in range(N_DEV - 1):
            slot = s % 2
            nxt_r = partial(lax.rem(my + (N_DEV - 2 - s), N_DEV), 0, nh)
            nxt_l = partial(lax.rem(my + 2 + s, N_DEV), nh, n)

            for q, ring in enumerate(rings):
                nxt = nxt_r if q < 2 else nxt_l
                lo = ring["col0"] - (0 if q < 2 else nh)
                sub = nxt[:, lo:lo + N_SUB]

                ring["rdmas"][s].wait_recv()
                if s < N_DEV - 2:
                    if s >= 1:
                        ring["rdmas"][s - 1].wait_send()
                    ring["send_buf"][(s + 1) % 2, :, :] = (
                        ring["comm_buf"][slot, :, :] + sub
                    )
                    start_hop(ring, s + 1)
                else:
                    c0 = ring["col0"]
                    out_ref[:, c0:c0 + N_SUB] = (
                        ring["comm_buf"][slot, :, :] + sub
                    )
                if s <= N_DEV - 4:
                    pl.semaphore_signal(
                        ring["credit"], inc=1,
                        device_id=(ring["upstream"],),
                        device_id_type=pl.DeviceIdType.MESH,
                    )

        for ring in rings:
            ring["rdmas"][N_DEV - 3].wait_send()
            ring["rdmas"][N_DEV - 2].wait_send()

    return pl.pallas_call(
        body,
        out_shape=jax.ShapeDtypeStruct((M_PER, n), jnp.float32),
        in_specs=[
            pl.BlockSpec(memory_space=pltpu.VMEM),
            pl.BlockSpec(memory_space=pltpu.VMEM),
        ],
        out_specs=pl.BlockSpec(memory_space=pltpu.VMEM),
        scratch_shapes=(
            [pltpu.VMEM((2, M_PER, N_SUB), jnp.float32)] * 4
            + [pltpu.VMEM((2, M_PER, N_SUB), jnp.float32)] * 4
            + [pltpu.SemaphoreType.DMA((2,))] * 4
            + [pltpu.SemaphoreType.DMA((2,))] * 4
            + [pltpu.SemaphoreType.REGULAR] * 4
        ),
        compiler_params=pltpu.CompilerParams(collective_id=0),
    )(x, w_mat)


# device time: 181810 ns/iter; 2.0726x vs baseline; 1.0107x over previous
import jax
import jax.numpy as jnp
from jax import lax
from jax.experimental import pallas as pl
from jax.experimental.pallas import tpu as pltpu

N_DEV = 16
M_PER = 256
N_SUB = 512


def kernel(x, w_mat):
    m_tot, k_per = x.shape
    _, n = w_mat.shape
    nh = n // 2

    def body(x_ref, w_ref, out_ref, *scratch):
        my = lax.axis_index("i")
        left = lax.rem(my + N_DEV - 1, N_DEV)
        right = lax.rem(my + 1, N_DEV)

        rings = []
        for q, (is_right, h) in enumerate(
            [(True, 0), (False, 0), (True, 1), (False, 1)]
        ):
            rings.append(dict(
                send_buf=scratch[q],
                comm_buf=scratch[4 + q],
                send_sems=scratch[8 + q],
                recv_sems=scratch[12 + q],
                credit=scratch[16 + q],
                dst=right if is_right else left,
                upstream=left if is_right else right,
                is_right=is_right,
                col0=(0 if is_right else nh) + h * N_SUB,
                rdmas=[],
            ))

        barrier_sem = pltpu.get_barrier_semaphore()
        for nbr in (left, right):
            pl.semaphore_signal(
                barrier_sem, inc=1,
                device_id=(nbr,), device_id_type=pl.DeviceIdType.MESH,
            )

        def partial(c, lo, hi):
            rows = x_ref[pl.ds(c * M_PER, M_PER), :]
            return jnp.dot(rows, w_ref[:, lo:hi],
                           preferred_element_type=jnp.float32)

        def start_hop(ring, s):
            slot = s % 2
            if s >= 2:
                pl.semaphore_wait(ring["credit"], 1)
            rdma = pltpu.make_async_remote_copy(
                src_ref=ring["send_buf"].at[slot],
                dst_ref=ring["comm_buf"].at[slot],
                send_sem=ring["send_sems"].at[slot],
                recv_sem=ring["recv_sems"].at[slot],
                device_id=(ring["dst"],),
                device_id_type=pl.DeviceIdType.MESH,
            )
            rdma.start()
            ring["rdmas"].append(rdma)

        c_seed_r = lax.rem(my + N_DEV - 1, N_DEV)
        c_seed_l = lax.rem(my + 1, N_DEV)
        for h in (0, 1):
            for ring in rings[2 * h:2 * h + 2]:
                c = c_seed_r if ring["is_right"] else c_seed_l
                ring["send_buf"][0, :, :] = partial(
                    c, ring["col0"], ring["col0"] + N_SUB
                )
            if h == 0:
                pl.semaphore_wait(barrier_sem, 2)
            for ring in rings[2 * h:2 * h + 2]:
                start_hop(ring, 0)

        for s in range(N_DEV - 1):
            slot = s % 2
            nxt_r = partial(lax.rem(my + (N_DEV - 2 - s), N_DEV), 0, nh)
            nxt_l = partial(lax.rem(my + 2 + s, N_DEV), nh, n)

            for ring in rings:
                nxt = nxt_r if ring["is_right"] else nxt_l
                lo = ring["col0"] - (0 if ring["is_right"] else nh)
                sub = nxt[:, lo:lo + N_SUB]

                ring["rdmas"][s].wait_recv()
                if s < N_DEV - 2:
                    if s >= 1:
                        ring["rdmas"][s - 1].wait_send()
                    ring["send_buf"][(s + 1) % 2, :, :] = (
                        ring["comm_buf"][slot, :, :] + sub
                    )
                    start_hop(ring, s + 1)
                else:
                    c0 = ring["col0"]
                    out_ref[:, c0:c0 + N_SUB] = (
                        ring["comm_buf"][slot, :, :] + sub
                    )
                if s <= N_DEV - 4:
                    pl.semaphore_signal(
                        ring["credit"], inc=1,
                        device_id=(ring["upstream"],),
                        device_id_type=pl.DeviceIdType.MESH,
                    )

        for ring in rings:
            ring["rdmas"][N_DEV - 3].wait_send()
            ring["rdmas"][N_DEV - 2].wait_send()

    return pl.pallas_call(
        body,
        out_shape=jax.ShapeDtypeStruct((M_PER, n), jnp.float32),
        in_specs=[
            pl.BlockSpec(memory_space=pltpu.VMEM),
            pl.BlockSpec(memory_space=pltpu.VMEM),
        ],
        out_specs=pl.BlockSpec(memory_space=pltpu.VMEM),
        scratch_shapes=(
            [pltpu.VMEM((2, M_PER, N_SUB), jnp.float32)] * 4
            + [pltpu.VMEM((2, M_PER, N_SUB), jnp.float32)] * 4
            + [pltpu.SemaphoreType.DMA((2,))] * 4
            + [pltpu.SemaphoreType.DMA((2,))] * 4
            + [pltpu.SemaphoreType.REGULAR] * 4
        ),
        compiler_params=pltpu.CompilerParams(collective_id=0),
    )(x, w_mat)
